# Initial kernel scaffold; baseline (speedup 1.0000x reference)
import jax
import jax.numpy as jnp
from jax.experimental import pallas as pl


def kernel(x, segment_ids):
    raise NotImplementedError("write your pallas kernel here")


# speedup vs baseline: 5.5506x; 5.5506x over previous
import jax
import jax.numpy as jnp
from jax import lax
from jax.experimental import pallas as pl
from jax.experimental.pallas import tpu as pltpu
import jax.experimental.pallas.tpu_sc as plsc

NR = 320000
D = 128
S = 10000
NC = 2
NS = 16
NW = NC * NS
RPT = NR // NW
B = 80
NB = RPT // B
SP = 10240
CH = SP // NS


def _sc_body(x_hbm, ids_hbm, sums_hbm, counts_hbm,
             ids_v, xbuf0, xbuf1, ones_v, zcnt_v, acc_sh, cnt_sh):
    c = lax.axis_index("c")
    s = lax.axis_index("s")
    w = c * NS + s

    zeros16 = jnp.zeros((16,), jnp.float32)
    for k in range(B // 16):
        ones_v[pl.ds(k * 16, 16)] = jnp.ones((16,), jnp.float32)

    def zrow(i, carry):
        for k in range(D // 16):
            xbuf0[i, pl.ds(k * 16, 16)] = zeros16
        return carry

    lax.fori_loop(0, B, zrow, 0)

    def zc(i, carry):
        zcnt_v[pl.ds(i * 16, 16)] = zeros16
        return carry

    lax.fori_loop(0, CH // 16, zc, 0)

    for k in range(CH // B):
        pltpu.sync_copy(xbuf0, acc_sh.at[pl.ds(s * CH + k * B, B), :])
    pltpu.sync_copy(zcnt_v, cnt_sh.at[pl.ds(s * CH, CH)])
    plsc.subcore_barrier()

    pltpu.sync_copy(ids_hbm.at[w], ids_v)

    base = w * RPT

    def batch(j, carry):
        row = base + j * B
        pltpu.sync_copy(x_hbm.at[pl.ds(row, B), :], xbuf0)
        pltpu.sync_copy(xbuf0, acc_sh.at[ids_v.at[j]], add=True)
        pltpu.sync_copy(ones_v, cnt_sh.at[ids_v.at[j]], add=True)
        return carry

    lax.fori_loop(0, NB, batch, 0)
    plsc.subcore_barrier()

    for k in range(CH // B):
        r0 = s * CH + k * B
        pltpu.sync_copy(acc_sh.at[pl.ds(r0, B), :], xbuf1)
        pltpu.sync_copy(xbuf1, sums_hbm.at[c, pl.ds(r0, B), :])
    pltpu.sync_copy(cnt_sh.at[pl.ds(s * CH, CH)], zcnt_v)
    pltpu.sync_copy(zcnt_v, counts_hbm.at[pl.ds(c * SP + s * CH, CH)])


def _sc_segment_sum(x, ids3):
    f = pl.kernel(
        _sc_body,
        out_type=(jax.ShapeDtypeStruct((NC, SP, D), jnp.float32),
                  jax.ShapeDtypeStruct((NC * SP,), jnp.float32)),
        mesh=plsc.VectorSubcoreMesh(core_axis_name="c", subcore_axis_name="s"),
        scratch_types=(
            pltpu.VMEM((NB, B), jnp.int32),
            pltpu.VMEM((B, D), jnp.float32),
            pltpu.VMEM((B, D), jnp.float32),
            pltpu.VMEM((B,), jnp.float32),
            pltpu.VMEM((CH,), jnp.float32),
            pltpu.VMEM_SHARED((SP, D), jnp.float32),
            pltpu.VMEM_SHARED((NS * CH,), jnp.float32),
        ),
    )
    return f(x, ids3)


def _finish_body(sums_ref, counts_ref, out_ref):
    sm = sums_ref[0] + sums_ref[1]
    ct = counts_ref[0] + counts_ref[1]
    nodes = sm / jnp.maximum(ct, 1.0)
    isn = jnp.isnan(nodes)
    n_ok = jnp.sum(jnp.where(isn, 0.0, 1.0))
    mean_val = jnp.sum(jnp.where(isn, 0.0, nodes)) / n_ok
    out_ref[...] = jnp.where(isn, mean_val, nodes)


def kernel(x, segment_ids):
    ids3 = segment_ids.astype(jnp.int32).reshape(NW, NB, B)
    sums_p, counts_p = _sc_segment_sum(x, ids3)
    sums = sums_p[:, :S, :]
    counts3 = counts_p.reshape(NC, SP)[:, :S].reshape(NC, S, 1)
    return pl.pallas_call(
        _finish_body,
        out_shape=jax.ShapeDtypeStruct((S, D), jnp.float32),
    )(sums, counts3)
